# Initial kernel scaffold; baseline (speedup 1.0000x reference)
import jax
import jax.numpy as jnp
from jax.experimental import pallas as pl


def kernel(x, edge_index, batch, Wl1, bl1, Wr1, Wl2, bl2, Wr2, W1, b1, W2, b2, W3, b3):
    raise NotImplementedError("write your pallas kernel here")


# speedup vs baseline: 2.9368x; 2.9368x over previous
import functools

import jax
import jax.numpy as jnp
from jax import lax
from jax.experimental import pallas as pl
from jax.experimental.pallas import tpu as pltpu
from jax.experimental.pallas import tpu_sc as plsc

N = 10000
E = 160000
D = 256
G = 64
C = 10

NC = 2
NS = 16
DH = D // NC
EP = E // NS
K = 80
NCHUNK = EP // K
NPAD = 10240
TSLAB = NPAD // NS
KC = 40
EPC = E // (NC * NS)
NCHUNKC = EPC // KC
BLK = 1000
NBLK = N // BLK


def _make_sc_cnt():
    out_type = (jax.ShapeDtypeStruct((NPAD, DH), jnp.float32),
                jax.ShapeDtypeStruct((NPAD, DH), jnp.float32))

    scratch = dict(
        acc_cnt=pltpu.VMEM_SHARED((NPAD, DH), jnp.float32),
        idx_d=pltpu.VMEM((1, KC), jnp.int32),
        ones=pltpu.VMEM((KC, DH), jnp.float32),
    )

    def body(dst, orow, c0_out, c1_out, **scr):
        acc_cnt = scr["acc_cnt"]
        idx_d, ones = scr["idx_d"], scr["ones"]

        cid = lax.axis_index("c")
        tid = lax.axis_index("s")

        r0 = tid * TSLAB

        def zero_cnt(cch, _):
            zb = r0 + cch * KC
            pltpu.sync_copy(orow.at[pl.ds(KC, KC)], ones)
            pltpu.sync_copy(ones, acc_cnt.at[pl.ds(zb, KC)])
            return 0
        lax.fori_loop(0, TSLAB // KC, zero_cnt, 0)
        pltpu.sync_copy(orow.at[pl.ds(0, KC)], ones)

        plsc.subcore_barrier()

        def chunk(j, _):
            base = cid * (E // NC) + tid * EPC + j * KC
            pltpu.sync_copy(dst.at[pl.ds(base, KC)], idx_d.at[0])
            pltpu.sync_copy(ones, acc_cnt.at[idx_d.at[0]], add=True)
            return 0

        lax.fori_loop(0, NCHUNKC, chunk, 0)

        plsc.subcore_barrier()

        @pl.when(cid == 0)
        def _():
            pltpu.sync_copy(acc_cnt.at[pl.ds(r0, TSLAB)],
                            c0_out.at[pl.ds(r0, TSLAB)])

        @pl.when(cid == 1)
        def _():
            pltpu.sync_copy(acc_cnt.at[pl.ds(r0, TSLAB)],
                            c1_out.at[pl.ds(r0, TSLAB)])

    mesh = plsc.VectorSubcoreMesh(core_axis_name="c", subcore_axis_name="s",
                                  num_cores=NC, num_subcores=NS)
    return pl.kernel(body, out_type=out_type, mesh=mesh,
                     scratch_types=scratch)


def _make_sc_agg():
    out_type = (jax.ShapeDtypeStruct((NPAD, DH), jnp.float32),
                jax.ShapeDtypeStruct((NPAD, DH), jnp.float32))

    scratch = dict(
        acc=pltpu.VMEM_SHARED((NPAD, DH), jnp.float32),
        idx_s=pltpu.VMEM((1, K), jnp.int32),
        idx_d=pltpu.VMEM((1, K), jnp.int32),
        rows=pltpu.VMEM((K, DH), jnp.float32),
    )

    def body(xl0, xl1, src, dst, zrow, s0_out, s1_out, **scr):
        acc = scr["acc"]
        idx_s, idx_d = scr["idx_s"], scr["idx_d"]
        rows = scr["rows"]

        cid = lax.axis_index("c")
        tid = lax.axis_index("s")

        r0 = tid * TSLAB

        def zero_acc(cch, _):
            zb = r0 + cch * K
            pltpu.sync_copy(zrow.at[pl.ds(zb, K)], rows)
            pltpu.sync_copy(rows, acc.at[pl.ds(zb, K)])
            return 0
        lax.fori_loop(0, TSLAB // K, zero_acc, 0)

        plsc.subcore_barrier()

        def chunk(j, _):
            base = tid * EP + j * K
            pltpu.sync_copy(src.at[pl.ds(base, K)], idx_s.at[0])
            pltpu.sync_copy(dst.at[pl.ds(base, K)], idx_d.at[0])

            @pl.when(cid == 0)
            def _():
                pltpu.sync_copy(xl0.at[idx_s.at[0]], rows)

            @pl.when(cid == 1)
            def _():
                pltpu.sync_copy(xl1.at[idx_s.at[0]], rows)

            pltpu.sync_copy(rows, acc.at[idx_d.at[0]], add=True)
            return 0

        lax.fori_loop(0, NCHUNK, chunk, 0)

        plsc.subcore_barrier()

        @pl.when(cid == 0)
        def _():
            pltpu.sync_copy(acc.at[pl.ds(r0, TSLAB)],
                            s0_out.at[pl.ds(r0, TSLAB)])

        @pl.when(cid == 1)
        def _():
            pltpu.sync_copy(acc.at[pl.ds(r0, TSLAB)],
                            s1_out.at[pl.ds(r0, TSLAB)])

    mesh = plsc.VectorSubcoreMesh(core_axis_name="c", subcore_axis_name="s",
                                  num_cores=NC, num_subcores=NS)
    return pl.kernel(body, out_type=out_type, mesh=mesh,
                     scratch_types=scratch)


@functools.lru_cache(maxsize=None)
def _sc_cached(kind: str):
    return _make_sc_cnt() if kind == "cnt" else _make_sc_agg()


def _dotT(a, w):
    return lax.dot_general(a, w, (((1,), (1,)), ((), ())),
                           preferred_element_type=jnp.float32)


def _k1_body(x_ref, wl_ref, out0_ref, out1_ref):
    y = _dotT(x_ref[...], wl_ref[...])
    out0_ref[...] = y[:, :DH]
    out1_ref[...] = y[:, DH:]


def _k1(x, wl1):
    return pl.pallas_call(
        _k1_body,
        grid=(NBLK,),
        in_specs=[
            pl.BlockSpec((BLK, D), lambda i: (i, 0)),
            pl.BlockSpec((D, D), lambda i: (0, 0)),
        ],
        out_specs=[
            pl.BlockSpec((BLK, DH), lambda i: (i, 0)),
            pl.BlockSpec((BLK, DH), lambda i: (i, 0)),
        ],
        out_shape=[
            jax.ShapeDtypeStruct((N, DH), jnp.float32),
            jax.ShapeDtypeStruct((N, DH), jnp.float32),
        ],
    )(x, wl1)


def _k2_body(s0_ref, s1_ref, cnt_ref, x_ref, wr_ref, bl_ref, wl2_ref,
             h_ref, xl20_ref, xl21_ref):
    s = jnp.concatenate([s0_ref[...], s1_ref[...]], axis=1)
    recip = 1.0 / jnp.maximum(cnt_ref[:, 0:1], 1.0)
    h = s * recip + _dotT(x_ref[...], wr_ref[...]) + bl_ref[...]
    h = jnp.maximum(h, 0.0)
    h_ref[...] = h
    y2 = _dotT(h, wl2_ref[...])
    xl20_ref[...] = y2[:, :DH]
    xl21_ref[...] = y2[:, DH:]


def _k2(s0, s1, cntw, x, wr1, bl1, wl2):
    return pl.pallas_call(
        _k2_body,
        grid=(NBLK,),
        in_specs=[
            pl.BlockSpec((BLK, DH), lambda i: (i, 0)),
            pl.BlockSpec((BLK, DH), lambda i: (i, 0)),
            pl.BlockSpec((BLK, 16), lambda i: (i, 0)),
            pl.BlockSpec((BLK, D), lambda i: (i, 0)),
            pl.BlockSpec((D, D), lambda i: (0, 0)),
            pl.BlockSpec((1, D), lambda i: (0, 0)),
            pl.BlockSpec((D, D), lambda i: (0, 0)),
        ],
        out_specs=[
            pl.BlockSpec((BLK, D), lambda i: (i, 0)),
            pl.BlockSpec((BLK, DH), lambda i: (i, 0)),
            pl.BlockSpec((BLK, DH), lambda i: (i, 0)),
        ],
        out_shape=[
            jax.ShapeDtypeStruct((N, D), jnp.float32),
            jax.ShapeDtypeStruct((N, DH), jnp.float32),
            jax.ShapeDtypeStruct((N, DH), jnp.float32),
        ],
    )(s0, s1, cntw, x, wr1, bl1, wl2)


def _k3_body(s0_ref, s1_ref, cnt_ref, h1_ref, wr_ref, bl_ref, batch_ref,
             w1_ref, b1_ref, w2_ref, b2_ref, w3_ref, b3_ref,
             out_ref, pool_ref):
    i = pl.program_id(0)

    @pl.when(i == 0)
    def _():
        pool_ref[...] = jnp.zeros((G, D), jnp.float32)

    s = jnp.concatenate([s0_ref[...], s1_ref[...]], axis=1)
    recip = 1.0 / jnp.maximum(cnt_ref[:, 0:1], 1.0)
    h = s * recip + _dotT(h1_ref[...], wr_ref[...]) + bl_ref[...]
    h = jnp.maximum(h, 0.0)

    bcol = batch_ref[...]
    g0 = batch_ref[0, 0]
    g1 = batch_ref[BLK - 1, 0]

    def upd(g, _):
        mask = bcol == g
        contrib = jnp.max(jnp.where(mask, h, 0.0), axis=0, keepdims=True)
        cur = pool_ref[pl.ds(g, 1), :]
        pool_ref[pl.ds(g, 1), :] = jnp.maximum(cur, contrib)
        return 0

    lax.fori_loop(g0, g1 + 1, upd, 0)

    @pl.when(i == NBLK - 1)
    def _():
        p = pool_ref[...]
        a = jnp.maximum(_dotT(p, w1_ref[...]) + b1_ref[...], 0.0)
        a = jnp.maximum(_dotT(a, w2_ref[...]) + b2_ref[...], 0.0)
        lg = _dotT(a, w3_ref[...]) + b3_ref[...]
        m = jnp.max(lg, axis=1, keepdims=True)
        lse = m + jnp.log(jnp.sum(jnp.exp(lg - m), axis=1, keepdims=True))
        out_ref[...] = lg - lse


def _k3(s0, s1, cntw, h1, wr2, bl2, batchc, w1, b1, w2, b2, w3, b3):
    return pl.pallas_call(
        _k3_body,
        grid=(NBLK,),
        in_specs=[
            pl.BlockSpec((BLK, DH), lambda i: (i, 0)),
            pl.BlockSpec((BLK, DH), lambda i: (i, 0)),
            pl.BlockSpec((BLK, 16), lambda i: (i, 0)),
            pl.BlockSpec((BLK, D), lambda i: (i, 0)),
            pl.BlockSpec((D, D), lambda i: (0, 0)),
            pl.BlockSpec((1, D), lambda i: (0, 0)),
            pl.BlockSpec((BLK, 1), lambda i: (i, 0)),
            pl.BlockSpec((128, D), lambda i: (0, 0)),
            pl.BlockSpec((1, 128), lambda i: (0, 0)),
            pl.BlockSpec((32, 128), lambda i: (0, 0)),
            pl.BlockSpec((1, 32), lambda i: (0, 0)),
            pl.BlockSpec((C, 32), lambda i: (0, 0)),
            pl.BlockSpec((1, C), lambda i: (0, 0)),
        ],
        out_specs=pl.BlockSpec((G, C), lambda i: (0, 0)),
        out_shape=jax.ShapeDtypeStruct((G, C), jnp.float32),
        scratch_shapes=[pltpu.VMEM((G, D), jnp.float32)],
    )(s0, s1, cntw, h1, wr2, bl2, batchc, w1, b1, w2, b2, w3, b3)


def kernel(x, edge_index, batch, Wl1, bl1, Wr1, Wl2, bl2, Wr2,
           W1, b1, W2, b2, W3, b3):
    src = edge_index[0]
    dst = edge_index[1]
    batchc = batch.reshape(N, 1)

    zrow = jnp.zeros((NPAD, DH), jnp.float32)
    orow = jnp.concatenate([jnp.ones((KC, DH), jnp.float32),
                            jnp.zeros((KC, DH), jnp.float32)], axis=0)

    xl10, xl11 = _k1(x, Wl1)
    c0, c1 = _sc_cached("cnt")(dst, orow)
    cntw = (c0[:N, :16] + c1[:N, :16])
    s10p, s11p = _sc_cached("agg")(xl10, xl11, src, dst, zrow)
    h1, xl20, xl21 = _k2(s10p[:N], s11p[:N], cntw, x, Wr1,
                         bl1.reshape(1, D), Wl2)
    s20p, s21p = _sc_cached("agg")(xl20, xl21, src, dst, zrow)
    logits = _k3(s20p[:N], s21p[:N], cntw, h1, Wr2, bl2.reshape(1, D),
                 batchc, W1, b1.reshape(1, 128), W2, b2.reshape(1, 32),
                 W3, b3.reshape(1, C))
    return logits


# speedup vs baseline: 4.1430x; 1.4107x over previous
import functools

import jax
import jax.numpy as jnp
from jax import lax
from jax.experimental import pallas as pl
from jax.experimental.pallas import tpu as pltpu
from jax.experimental.pallas import tpu_sc as plsc

N = 10000
E = 160000
D = 256
G = 64
C = 10

NC = 2
NS = 16
DH = D // NC
EP = E // NS
K = 80
NCHUNK = EP // K
NPAD = 10240
TSLAB = NPAD // NS
KC = 40
EPC = E // (NC * NS)
NCHUNKC = EPC // KC
BLK = 1000
NBLK = N // BLK


def _make_sc_cnt():
    out_type = (jax.ShapeDtypeStruct((NPAD, DH), jnp.float32),
                jax.ShapeDtypeStruct((NPAD, DH), jnp.float32))

    scratch = dict(
        acc_cnt=pltpu.VMEM_SHARED((NPAD, DH), jnp.float32),
        idx_d=pltpu.VMEM((1, KC), jnp.int32),
        ones=pltpu.VMEM((KC, DH), jnp.float32),
    )

    def body(dst, orow, c0_out, c1_out, **scr):
        acc_cnt = scr["acc_cnt"]
        idx_d, ones = scr["idx_d"], scr["ones"]

        cid = lax.axis_index("c")
        tid = lax.axis_index("s")

        r0 = tid * TSLAB

        def zero_cnt(cch, _):
            zb = r0 + cch * KC
            pltpu.sync_copy(orow.at[pl.ds(KC, KC)], ones)
            pltpu.sync_copy(ones, acc_cnt.at[pl.ds(zb, KC)])
            return 0
        lax.fori_loop(0, TSLAB // KC, zero_cnt, 0)
        pltpu.sync_copy(orow.at[pl.ds(0, KC)], ones)

        plsc.subcore_barrier()

        def chunk(j, _):
            base = cid * (E // NC) + tid * EPC + j * KC
            pltpu.sync_copy(dst.at[pl.ds(base, KC)], idx_d.at[0])
            pltpu.sync_copy(ones, acc_cnt.at[idx_d.at[0]], add=True)
            return 0

        lax.fori_loop(0, NCHUNKC, chunk, 0)

        plsc.subcore_barrier()

        @pl.when(cid == 0)
        def _():
            pltpu.sync_copy(acc_cnt.at[pl.ds(r0, TSLAB)],
                            c0_out.at[pl.ds(r0, TSLAB)])

        @pl.when(cid == 1)
        def _():
            pltpu.sync_copy(acc_cnt.at[pl.ds(r0, TSLAB)],
                            c1_out.at[pl.ds(r0, TSLAB)])

    mesh = plsc.VectorSubcoreMesh(core_axis_name="c", subcore_axis_name="s",
                                  num_cores=NC, num_subcores=NS)
    return pl.kernel(body, out_type=out_type, mesh=mesh,
                     scratch_types=scratch)


def _make_sc_agg():
    out_type = (jax.ShapeDtypeStruct((NPAD, DH), jnp.float32),
                jax.ShapeDtypeStruct((NPAD, DH), jnp.float32))

    scratch = dict(
        acc=pltpu.VMEM_SHARED((NPAD, DH), jnp.float32),
        idx_sa=pltpu.VMEM((1, K), jnp.int32),
        idx_da=pltpu.VMEM((1, K), jnp.int32),
        rows_a=pltpu.VMEM((K, DH), jnp.float32),
        idx_sb=pltpu.VMEM((1, K), jnp.int32),
        idx_db=pltpu.VMEM((1, K), jnp.int32),
        rows_b=pltpu.VMEM((K, DH), jnp.float32),
        sem_a=pltpu.SemaphoreType.DMA,
        sem_b=pltpu.SemaphoreType.DMA,
    )

    def body(xl0, xl1, src, dst, zrow, s0_out, s1_out, **scr):
        acc = scr["acc"]
        bufs = ((scr["idx_sa"], scr["idx_da"], scr["rows_a"], scr["sem_a"]),
                (scr["idx_sb"], scr["idx_db"], scr["rows_b"], scr["sem_b"]))

        cid = lax.axis_index("c")
        tid = lax.axis_index("s")

        r0 = tid * TSLAB
        rows0 = bufs[0][2]

        def zero_acc(cch, _):
            zb = r0 + cch * K
            pltpu.sync_copy(zrow.at[pl.ds(zb, K)], rows0)
            pltpu.sync_copy(rows0, acc.at[pl.ds(zb, K)])
            return 0
        lax.fori_loop(0, TSLAB // K, zero_acc, 0)

        plsc.subcore_barrier()

        def load_idx(j, b):
            base = tid * EP + j * K
            pltpu.sync_copy(src.at[pl.ds(base, K)], b[0].at[0])
            pltpu.sync_copy(dst.at[pl.ds(base, K)], b[1].at[0])

        def start_gather(b):
            @pl.when(cid == 0)
            def _():
                pltpu.async_copy(xl0.at[b[0].at[0]], b[2], b[3])

            @pl.when(cid == 1)
            def _():
                pltpu.async_copy(xl1.at[b[0].at[0]], b[2], b[3])

        def wait_gather(b):
            @pl.when(cid == 0)
            def _():
                pltpu.make_async_copy(xl0.at[b[0].at[0]], b[2], b[3]).wait()

            @pl.when(cid == 1)
            def _():
                pltpu.make_async_copy(xl1.at[b[0].at[0]], b[2], b[3]).wait()

        load_idx(0, bufs[0])
        start_gather(bufs[0])

        def chunk(j, _):
            def step(cur, nxt):
                @pl.when(j + 1 < NCHUNK)
                def _():
                    load_idx(j + 1, nxt)
                    start_gather(nxt)
                wait_gather(cur)
                pltpu.sync_copy(cur[2], acc.at[cur[1].at[0]], add=True)

            @pl.when(j % 2 == 0)
            def _():
                step(bufs[0], bufs[1])

            @pl.when(j % 2 == 1)
            def _():
                step(bufs[1], bufs[0])
            return 0

        lax.fori_loop(0, NCHUNK, chunk, 0)

        plsc.subcore_barrier()

        @pl.when(cid == 0)
        def _():
            pltpu.sync_copy(acc.at[pl.ds(r0, TSLAB)],
                            s0_out.at[pl.ds(r0, TSLAB)])

        @pl.when(cid == 1)
        def _():
            pltpu.sync_copy(acc.at[pl.ds(r0, TSLAB)],
                            s1_out.at[pl.ds(r0, TSLAB)])

    mesh = plsc.VectorSubcoreMesh(core_axis_name="c", subcore_axis_name="s",
                                  num_cores=NC, num_subcores=NS)
    return pl.kernel(body, out_type=out_type, mesh=mesh,
                     scratch_types=scratch)


@functools.lru_cache(maxsize=None)
def _sc_cached(kind: str):
    return _make_sc_cnt() if kind == "cnt" else _make_sc_agg()


def _dotT(a, w):
    return lax.dot_general(a, w, (((1,), (1,)), ((), ())),
                           preferred_element_type=jnp.float32)


def _k1_body(x_ref, wl_ref, out0_ref, out1_ref):
    y = _dotT(x_ref[...], wl_ref[...])
    out0_ref[...] = y[:, :DH]
    out1_ref[...] = y[:, DH:]


def _k1(x, wl1):
    return pl.pallas_call(
        _k1_body,
        grid=(NBLK,),
        in_specs=[
            pl.BlockSpec((BLK, D), lambda i: (i, 0)),
            pl.BlockSpec((D, D), lambda i: (0, 0)),
        ],
        out_specs=[
            pl.BlockSpec((BLK, DH), lambda i: (i, 0)),
            pl.BlockSpec((BLK, DH), lambda i: (i, 0)),
        ],
        out_shape=[
            jax.ShapeDtypeStruct((N, DH), jnp.float32),
            jax.ShapeDtypeStruct((N, DH), jnp.float32),
        ],
    )(x, wl1)


def _k2_body(s0_ref, s1_ref, cnt_ref, x_ref, wr_ref, bl_ref, wl2_ref,
             h_ref, xl20_ref, xl21_ref):
    s = jnp.concatenate([s0_ref[...], s1_ref[...]], axis=1)
    recip = 1.0 / jnp.maximum(cnt_ref[:, 0:1], 1.0)
    h = s * recip + _dotT(x_ref[...], wr_ref[...]) + bl_ref[...]
    h = jnp.maximum(h, 0.0)
    h_ref[...] = h
    y2 = _dotT(h, wl2_ref[...])
    xl20_ref[...] = y2[:, :DH]
    xl21_ref[...] = y2[:, DH:]


def _k2(s0, s1, cntw, x, wr1, bl1, wl2):
    return pl.pallas_call(
        _k2_body,
        grid=(NBLK,),
        in_specs=[
            pl.BlockSpec((BLK, DH), lambda i: (i, 0)),
            pl.BlockSpec((BLK, DH), lambda i: (i, 0)),
            pl.BlockSpec((BLK, 16), lambda i: (i, 0)),
            pl.BlockSpec((BLK, D), lambda i: (i, 0)),
            pl.BlockSpec((D, D), lambda i: (0, 0)),
            pl.BlockSpec((1, D), lambda i: (0, 0)),
            pl.BlockSpec((D, D), lambda i: (0, 0)),
        ],
        out_specs=[
            pl.BlockSpec((BLK, D), lambda i: (i, 0)),
            pl.BlockSpec((BLK, DH), lambda i: (i, 0)),
            pl.BlockSpec((BLK, DH), lambda i: (i, 0)),
        ],
        out_shape=[
            jax.ShapeDtypeStruct((N, D), jnp.float32),
            jax.ShapeDtypeStruct((N, DH), jnp.float32),
            jax.ShapeDtypeStruct((N, DH), jnp.float32),
        ],
    )(s0, s1, cntw, x, wr1, bl1, wl2)


def _k3_body(s0_ref, s1_ref, cnt_ref, h1_ref, wr_ref, bl_ref, batch_ref,
             w1_ref, b1_ref, w2_ref, b2_ref, w3_ref, b3_ref,
             out_ref, pool_ref):
    i = pl.program_id(0)

    @pl.when(i == 0)
    def _():
        pool_ref[...] = jnp.zeros((G, D), jnp.float32)

    s = jnp.concatenate([s0_ref[...], s1_ref[...]], axis=1)
    recip = 1.0 / jnp.maximum(cnt_ref[:, 0:1], 1.0)
    h = s * recip + _dotT(h1_ref[...], wr_ref[...]) + bl_ref[...]
    h = jnp.maximum(h, 0.0)

    bcol = batch_ref[...]
    g0 = batch_ref[0, 0]
    g1 = batch_ref[BLK - 1, 0]

    def upd(g, _):
        mask = bcol == g
        contrib = jnp.max(jnp.where(mask, h, 0.0), axis=0, keepdims=True)
        cur = pool_ref[pl.ds(g, 1), :]
        pool_ref[pl.ds(g, 1), :] = jnp.maximum(cur, contrib)
        return 0

    lax.fori_loop(g0, g1 + 1, upd, 0)

    @pl.when(i == NBLK - 1)
    def _():
        p = pool_ref[...]
        a = jnp.maximum(_dotT(p, w1_ref[...]) + b1_ref[...], 0.0)
        a = jnp.maximum(_dotT(a, w2_ref[...]) + b2_ref[...], 0.0)
        lg = _dotT(a, w3_ref[...]) + b3_ref[...]
        m = jnp.max(lg, axis=1, keepdims=True)
        lse = m + jnp.log(jnp.sum(jnp.exp(lg - m), axis=1, keepdims=True))
        out_ref[...] = lg - lse


def _k3(s0, s1, cntw, h1, wr2, bl2, batchc, w1, b1, w2, b2, w3, b3):
    return pl.pallas_call(
        _k3_body,
        grid=(NBLK,),
        in_specs=[
            pl.BlockSpec((BLK, DH), lambda i: (i, 0)),
            pl.BlockSpec((BLK, DH), lambda i: (i, 0)),
            pl.BlockSpec((BLK, 16), lambda i: (i, 0)),
            pl.BlockSpec((BLK, D), lambda i: (i, 0)),
            pl.BlockSpec((D, D), lambda i: (0, 0)),
            pl.BlockSpec((1, D), lambda i: (0, 0)),
            pl.BlockSpec((BLK, 1), lambda i: (i, 0)),
            pl.BlockSpec((128, D), lambda i: (0, 0)),
            pl.BlockSpec((1, 128), lambda i: (0, 0)),
            pl.BlockSpec((32, 128), lambda i: (0, 0)),
            pl.BlockSpec((1, 32), lambda i: (0, 0)),
            pl.BlockSpec((C, 32), lambda i: (0, 0)),
            pl.BlockSpec((1, C), lambda i: (0, 0)),
        ],
        out_specs=pl.BlockSpec((G, C), lambda i: (0, 0)),
        out_shape=jax.ShapeDtypeStruct((G, C), jnp.float32),
        scratch_shapes=[pltpu.VMEM((G, D), jnp.float32)],
    )(s0, s1, cntw, h1, wr2, bl2, batchc, w1, b1, w2, b2, w3, b3)


def kernel(x, edge_index, batch, Wl1, bl1, Wr1, Wl2, bl2, Wr2,
           W1, b1, W2, b2, W3, b3):
    src = edge_index[0]
    dst = edge_index[1]
    batchc = batch.reshape(N, 1)

    zrow = jnp.zeros((NPAD, DH), jnp.float32)
    orow = jnp.concatenate([jnp.ones((KC, DH), jnp.float32),
                            jnp.zeros((KC, DH), jnp.float32)], axis=0)

    xl10, xl11 = _k1(x, Wl1)
    c0, c1 = _sc_cached("cnt")(dst, orow)
    cntw = (c0[:N, :16] + c1[:N, :16])
    s10p, s11p = _sc_cached("agg")(xl10, xl11, src, dst, zrow)
    h1, xl20, xl21 = _k2(s10p[:N], s11p[:N], cntw, x, Wr1,
                         bl1.reshape(1, D), Wl2)
    s20p, s21p = _sc_cached("agg")(xl20, xl21, src, dst, zrow)
    logits = _k3(s20p[:N], s21p[:N], cntw, h1, Wr2, bl2.reshape(1, D),
                 batchc, W1, b1.reshape(1, 128), W2, b2.reshape(1, 32),
                 W3, b3.reshape(1, C))
    return logits
